# baseline (speedup 1.0000x reference)
import functools

import jax
import jax.numpy as jnp
from jax import lax
from jax.experimental import pallas as pl
from jax.experimental.pallas import tpu as pltpu
from jax.experimental.pallas import tpu_sc as plsc

IN_F = 512
OUT_F = 32
LANES = 16


def _build_mask_sc(idx, B, Q, M):
    mesh = plsc.VectorSubcoreMesh(core_axis_name="c", subcore_axis_name="s")

    @functools.partial(
        pl.kernel,
        mesh=mesh,
        out_type=jax.ShapeDtypeStruct((B, Q), jnp.float32),
        scratch_types=[
            pltpu.VMEM((M,), jnp.int32),
            pltpu.VMEM((Q,), jnp.float32),
        ],
        compiler_params=pltpu.CompilerParams(needs_layout_passes=False),
    )
    def mask_kernel(idx_hbm, mask_hbm, idx_v, mask_v):
        num_c = lax.axis_size("c")
        wid = lax.axis_index("s") * num_c + lax.axis_index("c")

        @pl.when(wid < B)
        def _():
            pltpu.sync_copy(idx_hbm.at[wid], idx_v)

            zeros = jnp.zeros((LANES,), jnp.float32)

            def zero_body(i, carry):
                mask_v[pl.ds(i * LANES, LANES)] = zeros
                return carry

            lax.fori_loop(0, Q // LANES, zero_body, 0)

            ones = jnp.ones((LANES,), jnp.float32)

            def scat_body(i, carry):
                ids = idx_v[pl.ds(i * LANES, LANES)]
                plsc.store_scatter(mask_v, [ids], ones)
                return carry

            lax.fori_loop(0, M // LANES, scat_body, 0)

            pltpu.sync_copy(mask_v, mask_hbm.at[wid])

    return mask_kernel(idx)


def _masked_linear_tc(d_A, mask3, WT, b2, B, Q):
    grid = (B,)

    def body(x_ref, m_ref, wt_ref, b_ref, o_ref):
        acc = jnp.dot(x_ref[0], wt_ref[...], preferred_element_type=jnp.float32)
        ones_row = jnp.ones((1, OUT_F), jnp.float32)
        mcol = lax.dot_general(
            m_ref[0],
            ones_row,
            (((0,), (0,)), ((), ())),
            preferred_element_type=jnp.float32,
        )
        o_ref[0] = (acc + b_ref[...]) * mcol

    return pl.pallas_call(
        body,
        grid=grid,
        in_specs=[
            pl.BlockSpec((1, Q, IN_F), lambda b: (b, 0, 0)),
            pl.BlockSpec((1, 1, Q), lambda b: (b, 0, 0)),
            pl.BlockSpec((IN_F, OUT_F), lambda b: (0, 0)),
            pl.BlockSpec((1, OUT_F), lambda b: (0, 0)),
        ],
        out_specs=pl.BlockSpec((1, Q, OUT_F), lambda b: (b, 0, 0)),
        out_shape=jax.ShapeDtypeStruct((B, Q, OUT_F), d_A.dtype),
    )(d_A, mask3, WT, b2)


def kernel(d_A, masked_indices_list, W, b):
    B, Q, _ = d_A.shape
    M = masked_indices_list.shape[1]
    idx = masked_indices_list.astype(jnp.int32)
    mask = _build_mask_sc(idx, B, Q, M)
    mask3 = mask.reshape(B, 1, Q)
    WT = W.T
    b2 = b.reshape(1, OUT_F)
    return _masked_linear_tc(d_A, mask3, WT, b2, B, Q)


# speedup vs baseline: 1.0034x; 1.0034x over previous
import functools

import jax
import jax.numpy as jnp
from jax import lax
from jax.experimental import pallas as pl
from jax.experimental.pallas import tpu as pltpu
from jax.experimental.pallas import tpu_sc as plsc

IN_F = 512
OUT_F = 32
LANES = 16


def _build_mask_sc(idx, B, Q, M):
    mesh = plsc.VectorSubcoreMesh(core_axis_name="c", subcore_axis_name="s")

    @functools.partial(
        pl.kernel,
        mesh=mesh,
        out_type=jax.ShapeDtypeStruct((B, Q), jnp.float32),
        scratch_types=[
            pltpu.VMEM((M,), jnp.int32),
            pltpu.VMEM((Q,), jnp.float32),
        ],
        compiler_params=pltpu.CompilerParams(needs_layout_passes=False),
    )
    def mask_kernel(idx_hbm, mask_hbm, idx_v, mask_v):
        num_c = lax.axis_size("c")
        wid = lax.axis_index("s") * num_c + lax.axis_index("c")

        @pl.when(wid < B)
        def _():
            pltpu.sync_copy(idx_hbm.at[wid], idx_v)

            zeros = jnp.zeros((LANES,), jnp.float32)

            def zero_body(i, carry):
                mask_v[pl.ds(i * LANES, LANES)] = zeros
                return carry

            lax.fori_loop(0, Q // LANES, zero_body, 0)

            ones = jnp.ones((LANES,), jnp.float32)

            def scat_body(i, carry):
                ids = idx_v[pl.ds(i * LANES, LANES)]
                plsc.store_scatter(mask_v, [ids], ones)
                return carry

            lax.fori_loop(0, M // LANES, scat_body, 0)

            pltpu.sync_copy(mask_v, mask_hbm.at[wid])

    return mask_kernel(idx)


def _masked_linear_tc(d_A, mask3, WT, b2, B, Q):
    grid = (B,)

    def body(x_ref, m_ref, wt_ref, b_ref, o_ref):
        xb = x_ref[0].astype(jnp.bfloat16)
        acc = jnp.dot(xb, wt_ref[...], preferred_element_type=jnp.float32)
        ones_row = jnp.ones((1, OUT_F), jnp.float32)
        mcol = lax.dot_general(
            m_ref[0],
            ones_row,
            (((0,), (0,)), ((), ())),
            preferred_element_type=jnp.float32,
        )
        o_ref[0] = (acc + b_ref[...]) * mcol

    return pl.pallas_call(
        body,
        grid=grid,
        in_specs=[
            pl.BlockSpec((1, Q, IN_F), lambda b: (b, 0, 0)),
            pl.BlockSpec((1, 1, Q), lambda b: (b, 0, 0)),
            pl.BlockSpec((IN_F, OUT_F), lambda b: (0, 0)),
            pl.BlockSpec((1, OUT_F), lambda b: (0, 0)),
        ],
        out_specs=pl.BlockSpec((1, Q, OUT_F), lambda b: (b, 0, 0)),
        out_shape=jax.ShapeDtypeStruct((B, Q, OUT_F), d_A.dtype),
    )(d_A, mask3, WT, b2)


def kernel(d_A, masked_indices_list, W, b):
    B, Q, _ = d_A.shape
    M = masked_indices_list.shape[1]
    idx = masked_indices_list.astype(jnp.int32)
    mask = _build_mask_sc(idx, B, Q, M)
    mask3 = mask.reshape(B, 1, Q)
    WT = W.T.astype(jnp.bfloat16)
    b2 = b.reshape(1, OUT_F)
    return _masked_linear_tc(d_A, mask3, WT, b2, B, Q)
